# baseline (speedup 1.0000x reference)
import functools

import jax
import jax.numpy as jnp
from jax import lax
from jax.experimental import pallas as pl
from jax.experimental.pallas import tpu as pltpu
from jax.experimental.pallas import tpu_sc as plsc

VOCAB = 1000000
EMBED = 32
B = 16384
L = 50

NC = 2
NS = 16
NW = NC * NS
B_PER_W = B // NW
BB = 16
NSTEP = B_PER_W // BB
CHUNK = BB * L


def _transpose_block(rows, outst):
    iota = lax.iota(jnp.int32, 16)
    jv0 = iota
    jv1 = iota + 16

    @plsc.parallel_loop(0, L, unroll=3)
    def tl(l):
        lv = jnp.full((16,), l, jnp.int32)
        for b_loc in range(BB):
            k = b_loc * L + l
            bv = jnp.full((16,), b_loc, jnp.int32)
            v0 = rows[k, pl.ds(0, 16)]
            v1 = rows[k, pl.ds(16, 16)]
            plsc.store_scatter(outst, [lv, jv0, bv], v0)
            plsc.store_scatter(outst, [lv, jv1, bv], v1)


@functools.partial(
    pl.kernel,
    out_type=jax.ShapeDtypeStruct((L, EMBED, B), jnp.float32),
    mesh=plsc.VectorSubcoreMesh(core_axis_name="c", subcore_axis_name="s"),
    scratch_types=[
        pltpu.VMEM((NSTEP, CHUNK), jnp.int32),
        pltpu.VMEM((CHUNK, EMBED), jnp.float32),
        pltpu.VMEM((CHUNK, EMBED), jnp.float32),
        pltpu.VMEM((L, EMBED, BB), jnp.float32),
        pltpu.VMEM((L, EMBED, BB), jnp.float32),
        pltpu.SemaphoreType.DMA,
        pltpu.SemaphoreType.DMA,
        pltpu.SemaphoreType.DMA,
        pltpu.SemaphoreType.DMA,
    ],
    compiler_params=pltpu.CompilerParams(
        use_tc_tiling_on_sc=False, needs_layout_passes=False),
)
def _gather_sc(idx_hbm, table_hbm, out_hbm, idx_v, rows0, rows1, outst0,
               outst1, gsem0, gsem1, osem0, osem1):
    wid = lax.axis_index("s") * NC + lax.axis_index("c")
    base_b = wid * B_PER_W

    pltpu.sync_copy(idx_hbm.at[wid], idx_v)

    rows = [rows0, rows1]
    gsems = [gsem0, gsem1]
    outst = [outst0, outst1]
    osems = [osem0, osem1]

    def out_dst(k):
        return out_hbm.at[:, :, pl.ds(base_b + k * BB, BB)]

    def step(k, b, first):
        pltpu.make_async_copy(table_hbm.at[idx_v.at[k]], rows[b], gsems[b]).wait()
        if not first:
            pltpu.make_async_copy(outst[b], out_dst(k - 2), osems[b]).wait()
        _transpose_block(rows[b], outst[b])
        pltpu.async_copy(outst[b], out_dst(k), osems[b])
        @pl.when(k + 2 < NSTEP)
        def _():
            pltpu.async_copy(table_hbm.at[idx_v.at[k + 2]], rows[b], gsems[b])

    pltpu.async_copy(table_hbm.at[idx_v.at[0]], rows[0], gsems[0])
    pltpu.async_copy(table_hbm.at[idx_v.at[1]], rows[1], gsems[1])

    step(0, 0, True)
    step(1, 1, True)

    def pair(t, c):
        k0 = t * 2
        step(k0, 0, False)
        step(k0 + 1, 1, False)
        return c

    lax.fori_loop(1, NSTEP // 2, pair, 0)

    pltpu.make_async_copy(outst[0], out_dst(NSTEP - 2), osems[0]).wait()
    pltpu.make_async_copy(outst[1], out_dst(NSTEP - 1), osems[1]).wait()


def kernel(x, embeddings):
    idx = x.reshape(NW, NSTEP, CHUNK)
    out_phys = _gather_sc(idx, embeddings)
    return out_phys.transpose(2, 0, 1)


# speedup vs baseline: 1.0125x; 1.0125x over previous
import functools

import jax
import jax.numpy as jnp
from jax import lax
from jax.experimental import pallas as pl
from jax.experimental.pallas import tpu as pltpu
from jax.experimental.pallas import tpu_sc as plsc

VOCAB = 1000000
EMBED = 32
B = 16384
L = 50

NC = 2
NS = 16
NW = NC * NS
B_PER_W = B // NW
BB = 16
NSTEP = B_PER_W // BB
CHUNK = BB * L


def _transpose_block(rows, outst):
    iota = lax.iota(jnp.int32, 16)
    jv0 = iota
    jv1 = iota + 16

    @plsc.parallel_loop(0, L, unroll=2)
    def tl(l):
        lv = jnp.full((16,), l, jnp.int32)
        for b_loc in range(BB):
            k = b_loc * L + l
            bv = jnp.full((16,), b_loc, jnp.int32)
            v0 = rows[k, pl.ds(0, 16)]
            v1 = rows[k, pl.ds(16, 16)]
            plsc.store_scatter(outst, [lv, jv0, bv], v0)
            plsc.store_scatter(outst, [lv, jv1, bv], v1)


@functools.partial(
    pl.kernel,
    out_type=jax.ShapeDtypeStruct((L, EMBED, B), jnp.float32),
    mesh=plsc.VectorSubcoreMesh(core_axis_name="c", subcore_axis_name="s"),
    scratch_types=[
        pltpu.VMEM((NSTEP, CHUNK), jnp.int32),
        pltpu.VMEM((CHUNK, EMBED), jnp.float32),
        pltpu.VMEM((CHUNK, EMBED), jnp.float32),
        pltpu.VMEM((L, EMBED, BB), jnp.float32),
        pltpu.VMEM((L, EMBED, BB), jnp.float32),
        pltpu.SemaphoreType.DMA,
        pltpu.SemaphoreType.DMA,
        pltpu.SemaphoreType.DMA,
        pltpu.SemaphoreType.DMA,
    ],
    compiler_params=pltpu.CompilerParams(
        use_tc_tiling_on_sc=False, needs_layout_passes=False),
)
def _gather_sc(idx_hbm, table_hbm, out_hbm, idx_v, rows0, rows1, outst0,
               outst1, gsem0, gsem1, osem0, osem1):
    wid = lax.axis_index("s") * NC + lax.axis_index("c")
    base_b = wid * B_PER_W

    pltpu.sync_copy(idx_hbm.at[wid], idx_v)

    rows = [rows0, rows1]
    gsems = [gsem0, gsem1]
    outst = [outst0, outst1]
    osems = [osem0, osem1]

    def out_dst(k):
        return out_hbm.at[:, :, pl.ds(base_b + k * BB, BB)]

    def step(k, b, first):
        pltpu.make_async_copy(table_hbm.at[idx_v.at[k]], rows[b], gsems[b]).wait()
        if not first:
            pltpu.make_async_copy(outst[b], out_dst(k - 2), osems[b]).wait()
        _transpose_block(rows[b], outst[b])
        pltpu.async_copy(outst[b], out_dst(k), osems[b])
        @pl.when(k + 2 < NSTEP)
        def _():
            pltpu.async_copy(table_hbm.at[idx_v.at[k + 2]], rows[b], gsems[b])

    pltpu.async_copy(table_hbm.at[idx_v.at[0]], rows[0], gsems[0])
    pltpu.async_copy(table_hbm.at[idx_v.at[1]], rows[1], gsems[1])

    step(0, 0, True)
    step(1, 1, True)

    def pair(t, c):
        k0 = t * 2
        step(k0, 0, False)
        step(k0 + 1, 1, False)
        return c

    lax.fori_loop(1, NSTEP // 2, pair, 0)

    pltpu.make_async_copy(outst[0], out_dst(NSTEP - 2), osems[0]).wait()
    pltpu.make_async_copy(outst[1], out_dst(NSTEP - 1), osems[1]).wait()


def kernel(x, embeddings):
    idx = x.reshape(NW, NSTEP, CHUNK)
    out_phys = _gather_sc(idx, embeddings)
    return out_phys.transpose(2, 0, 1)
